# baseline (speedup 1.0000x reference)
import functools

import jax
import jax.numpy as jnp
from jax import lax
from jax.experimental import pallas as pl
from jax.experimental.pallas import tpu as pltpu
from jax.experimental.pallas import tpu_sc as plsc

NUM_GRAPHS = 1024
MAX_DEPTH = 32
NW = 32


def _depthmax_sc_body(batch_hbm, depth_hbm, out_hbm, bch, dch, acc, *,
                      G, PER, CP, NG):
    wid = lax.axis_index("s") * 2 + lax.axis_index("c")
    base = wid * PER
    a = (base // 8) * 8
    lead = base - a
    pltpu.sync_copy(batch_hbm.at[pl.ds(a, CP)], bch)
    pltpu.sync_copy(depth_hbm.at[pl.ds(a, CP)], dch)

    neg1 = jnp.full((16,), -1, jnp.int32)

    def initb(j, c):
        for k in range(8):
            acc[pl.ds(j * 128 + k * 16, 16)] = neg1
        return c

    lax.fori_loop(0, G * 2 // 16, initb, 0)
    iota = lax.broadcasted_iota(jnp.int32, (16,), 0)
    stripe = iota * G

    def body(g, c):
        o = g * 16
        b = bch[pl.ds(o, 16)]
        d = dch[pl.ds(o, 16)]
        gi = o + iota
        valid = (gi >= lead) & (gi < lead + PER)
        bcl = jnp.clip(jnp.where(valid, b, 0), 0, G - 1)
        idx = stripe + bcl
        old = plsc.load_gather(acc, [idx], mask=valid)
        neww = jnp.maximum(old, jnp.where(valid, d, -1))
        plsc.store_scatter(acc, [idx], neww, mask=valid)
        return c

    lax.fori_loop(0, NG, body, 0)

    def mergeb(j, c):
        m = acc[pl.ds(j * 16, 16)]
        for l in range(1, 16):
            m = jnp.maximum(m, acc[pl.ds(l * G + j * 16, 16)])
        acc[pl.ds(j * 16, 16)] = m
        return c

    lax.fori_loop(0, G // 16, mergeb, 0)
    pltpu.sync_copy(acc.at[pl.ds(0, G)], out_hbm.at[wid])


def _depthmax_sc(batch_i, depth_i, G):
    N = batch_i.shape[0]
    PER = N // NW
    NG = (PER + 7 + 15) // 16
    CP = NG * 16
    amax = ((NW - 1) * PER // 8) * 8
    pad = amax + CP - N
    batch_p = jnp.concatenate(
        [batch_i, jnp.full((pad,), 1 << 20, jnp.int32)])
    depth_p = jnp.concatenate([depth_i, jnp.zeros((pad,), jnp.int32)])
    mesh = plsc.VectorSubcoreMesh(core_axis_name="c", subcore_axis_name="s")
    return pl.kernel(
        functools.partial(_depthmax_sc_body, G=G, PER=PER, CP=CP, NG=NG),
        out_type=jax.ShapeDtypeStruct((NW, G), jnp.int32),
        mesh=mesh,
        compiler_params=pltpu.CompilerParams(needs_layout_passes=False),
        scratch_types=[
            pltpu.VMEM((CP,), jnp.int32),
            pltpu.VMEM((CP,), jnp.int32),
            pltpu.VMEM((16 * G,), jnp.int32),
        ],
    )(batch_p, depth_p)


def _pick_block(n):
    for k in (4000, 2048, 2000, 1600, 1280, 1024, 1000, 800, 640, 512, 500, 400, 256, 200, 160, 128, 100, 80, 64, 50, 40, 32, 16, 8):
        if n % k == 0 and k % 8 == 0:
            return k
    return n


def _accum_body(g0_ref, g1_ref, h_ref, b_ref, mc_ref, w1_ref, b1_ref,
                w2_ref, b2_ref, acc_ref, *, G, K, L, W):
    i = pl.program_id(0)
    hb = h_ref[...]
    hb16 = hb.astype(jnp.bfloat16)
    hidden = jnp.tanh(
        jnp.dot(hb16, w1_ref[...],
                preferred_element_type=jnp.float32) + b1_ref[...])
    sc = (jnp.dot(hidden.astype(jnp.bfloat16), w2_ref[...],
                  preferred_element_type=jnp.float32)
          + b2_ref[...])
    mcol = mc_ref[...]
    brow = b_ref[0]

    e_cols = [jnp.exp(sc[:, j:j + 1]).astype(jnp.bfloat16) for j in range(4)]
    m16 = mcol.astype(jnp.bfloat16)
    pieces = [ec * hb16 for ec in e_cols] + [m16 * hb16] + e_cols + [m16]
    data = jnp.concatenate(pieces, axis=1)

    bmin = g0_ref[i]
    bmax = g1_ref[i]
    g0a = jnp.minimum((bmin // 8) * 8, G - L)
    ok = bmax < g0a + L

    @pl.when(i == 0)
    def _init():
        acc_ref[...] = jnp.zeros((G, W), jnp.float32)

    @pl.when(ok)
    def _local():
        lcol = jax.lax.broadcasted_iota(jnp.int32, (L, 1), 0) + g0a
        oh = (lcol == brow)
        contrib = jnp.dot(oh.astype(jnp.bfloat16), data,
                          preferred_element_type=jnp.float32)
        acc_ref[pl.ds(g0a, L), :] += contrib

    @pl.when(jnp.logical_not(ok))
    def _full():
        gcol = jax.lax.broadcasted_iota(jnp.int32, (G, 1), 0)
        oh = (gcol == brow)
        contrib = jnp.dot(oh.astype(jnp.bfloat16), data,
                          preferred_element_type=jnp.float32)
        acc_ref[...] += contrib


def _finish_body(acc_ref, dp_ref, pW_ref, pb_ref, de_ref, fW1_ref, fb1_ref,
                 fW2_ref, fb2_ref, g_ref, be_ref, out_ref, *, G, D):
    acc = acc_ref[...]
    heads = []
    for j in range(4):
        num = acc[:, j * D:(j + 1) * D]
        den = acc[:, 5 * D + j:5 * D + j + 1]
        den = jnp.where(den == 0.0, 1.0, den)
        heads.append(num / den)
    hcat = jnp.concatenate(heads, axis=1)
    hg = jnp.dot(hcat, pW_ref[...],
                 preferred_element_type=jnp.float32) + pb_ref[...]
    hb = acc[:, 4 * D:5 * D] / (acc[:, 5 * D + 4:5 * D + 5] + 1e-8)

    mdrow = jnp.max(dp_ref[...], axis=0, keepdims=True)
    mdrow = jnp.clip(jnp.maximum(mdrow, 0), 0, MAX_DEPTH - 1)
    kcol = jax.lax.broadcasted_iota(jnp.int32, (MAX_DEPTH, 1), 0)
    ohdT = (kcol == mdrow).astype(jnp.float32)
    denc = jax.lax.dot_general(
        ohdT, de_ref[...], (((0,), (0,)), ((), ())),
        preferred_element_type=jnp.float32)

    fused = jnp.concatenate([hg, hb, denc], axis=1)
    x = jnp.dot(fused, fW1_ref[...],
                preferred_element_type=jnp.float32) + fb1_ref[...]
    x = 0.5 * x * (1.0 + jax.lax.erf(x * 0.7071067811865476))
    x = jnp.dot(x, fW2_ref[...],
                preferred_element_type=jnp.float32) + fb2_ref[...]
    mu = jnp.mean(x, axis=-1, keepdims=True)
    var = jnp.mean((x - mu) ** 2, axis=-1, keepdims=True)
    out_ref[...] = (x - mu) / jnp.sqrt(var + 1e-5) * g_ref[...] + be_ref[...]


def kernel(h, batch, is_branch, depth, aW1, ab1, aW2, ab2, pW, pb, depth_embed,
           fW1, fb1, fW2, fb2, gamma, beta):
    N, D = h.shape
    H, _, dh = aW1.shape
    G = NUM_GRAPHS
    K = _pick_block(N)
    NB = N // K
    W = 5 * D + 5

    L = 128

    w1cat = jnp.transpose(aW1, (1, 0, 2)).reshape(D, H * dh).astype(jnp.bfloat16)
    b1row = ab1.reshape(1, H * dh)
    rows = jnp.arange(H * dh)
    w2p = jnp.zeros((H * dh, 128), jnp.float32).at[rows, rows // dh].set(
        aW2.reshape(H * dh)).astype(jnp.bfloat16)
    b2row = jnp.zeros((1, 128), jnp.float32).at[0, :H].set(ab2[:, 0])

    batch_i = batch.astype(jnp.int32)
    batch_r = batch_i.reshape(NB, 1, K)
    maskc = is_branch.astype(jnp.float32).reshape(N, 1)
    g0s = batch_i[0::K]
    g1s = batch_i[K - 1::K]

    dpart = _depthmax_sc(batch_i, depth.astype(jnp.int32), G)

    acc = pl.pallas_call(
        functools.partial(_accum_body, G=G, K=K, L=L, W=W),
        grid_spec=pltpu.PrefetchScalarGridSpec(
            num_scalar_prefetch=2,
            grid=(NB,),
            in_specs=[
                pl.BlockSpec((K, D), lambda i, *_: (i, 0)),
                pl.BlockSpec((1, 1, K), lambda i, *_: (i, 0, 0)),
                pl.BlockSpec((K, 1), lambda i, *_: (i, 0)),
                pl.BlockSpec((D, H * dh), lambda i, *_: (0, 0)),
                pl.BlockSpec((1, H * dh), lambda i, *_: (0, 0)),
                pl.BlockSpec((H * dh, 128), lambda i, *_: (0, 0)),
                pl.BlockSpec((1, 128), lambda i, *_: (0, 0)),
            ],
            out_specs=[
                pl.BlockSpec((G, W), lambda i, *_: (0, 0)),
            ],
        ),
        out_shape=[
            jax.ShapeDtypeStruct((G, W), jnp.float32),
        ],
    )(g0s, g1s, h, batch_r, maskc, w1cat, b1row, w2p, b2row)[0]

    dep_p = jnp.zeros((MAX_DEPTH, 128), jnp.float32).at[:, :depth_embed.shape[1]].set(depth_embed)
    fin = 2 * D + 128
    fW1p = jnp.zeros((fin, fW1.shape[1]), jnp.float32)
    fW1p = fW1p.at[:2 * D].set(fW1[:2 * D])
    fW1p = fW1p.at[2 * D:2 * D + depth_embed.shape[1]].set(fW1[2 * D:])

    out = pl.pallas_call(
        functools.partial(_finish_body, G=G, D=D),
        in_specs=[pl.BlockSpec(x.shape, lambda: tuple(0 for _ in x.shape))
                  for x in (acc, dpart, pW, pb.reshape(1, -1), dep_p, fW1p,
                            fb1.reshape(1, -1), fW2, fb2.reshape(1, -1),
                            gamma.reshape(1, -1), beta.reshape(1, -1))],
        out_specs=pl.BlockSpec((G, fW2.shape[1]), lambda: (0, 0)),
        out_shape=jax.ShapeDtypeStruct((G, fW2.shape[1]), jnp.float32),
    )(acc, dpart, pW, pb.reshape(1, -1), dep_p, fW1p, fb1.reshape(1, -1),
      fW2, fb2.reshape(1, -1), gamma.reshape(1, -1), beta.reshape(1, -1))
    return out


# speedup vs baseline: 1.0102x; 1.0102x over previous
import functools

import jax
import jax.numpy as jnp
from jax import lax
from jax.experimental import pallas as pl
from jax.experimental.pallas import tpu as pltpu
from jax.experimental.pallas import tpu_sc as plsc

NUM_GRAPHS = 1024
MAX_DEPTH = 32
NW = 32


def _depthmax_sc_body(batch_hbm, depth_hbm, out_hbm, bch, dch, acc, *,
                      G, PER, CP, NG):
    wid = lax.axis_index("s") * 2 + lax.axis_index("c")
    base = wid * PER
    a = (base // 8) * 8
    lead = base - a
    pltpu.sync_copy(batch_hbm.at[pl.ds(a, CP)], bch)
    pltpu.sync_copy(depth_hbm.at[pl.ds(a, CP)], dch)

    neg1 = jnp.full((16,), -1, jnp.int32)

    def initb(j, c):
        for k in range(8):
            acc[pl.ds(j * 128 + k * 16, 16)] = neg1
        return c

    lax.fori_loop(0, G * 2 // 16, initb, 0)
    iota = lax.broadcasted_iota(jnp.int32, (16,), 0)
    stripe = iota * G

    def body(g, c):
        o = g * 16
        b = bch[pl.ds(o, 16)]
        d = dch[pl.ds(o, 16)]
        gi = o + iota
        valid = (gi >= lead) & (gi < lead + PER)
        bcl = jnp.clip(jnp.where(valid, b, 0), 0, G - 1)
        idx = stripe + bcl
        old = plsc.load_gather(acc, [idx], mask=valid)
        neww = jnp.maximum(old, jnp.where(valid, d, -1))
        plsc.store_scatter(acc, [idx], neww, mask=valid)
        return c

    lax.fori_loop(0, NG, body, 0)

    def mergeb(j, c):
        m = acc[pl.ds(j * 16, 16)]
        for l in range(1, 16):
            m = jnp.maximum(m, acc[pl.ds(l * G + j * 16, 16)])
        acc[pl.ds(j * 16, 16)] = m
        return c

    lax.fori_loop(0, G // 16, mergeb, 0)
    pltpu.sync_copy(acc.at[pl.ds(0, G)], out_hbm.at[wid])


def _depthmax_sc(batch_i, depth_i, G):
    N = batch_i.shape[0]
    PER = N // NW
    NG = (PER + 7 + 15) // 16
    CP = NG * 16
    amax = ((NW - 1) * PER // 8) * 8
    pad = amax + CP - N
    batch_p = jnp.concatenate(
        [batch_i, jnp.full((pad,), 1 << 20, jnp.int32)])
    depth_p = jnp.concatenate([depth_i, jnp.zeros((pad,), jnp.int32)])
    mesh = plsc.VectorSubcoreMesh(core_axis_name="c", subcore_axis_name="s")
    return pl.kernel(
        functools.partial(_depthmax_sc_body, G=G, PER=PER, CP=CP, NG=NG),
        out_type=jax.ShapeDtypeStruct((NW, G), jnp.int32),
        mesh=mesh,
        compiler_params=pltpu.CompilerParams(needs_layout_passes=False),
        scratch_types=[
            pltpu.VMEM((CP,), jnp.int32),
            pltpu.VMEM((CP,), jnp.int32),
            pltpu.VMEM((16 * G,), jnp.int32),
        ],
    )(batch_p, depth_p)


def _pick_block(n):
    for k in (5000, 4000, 2048, 2000, 1600, 1280, 1024, 1000, 800, 640, 512, 500, 400, 256, 200, 160, 128, 100, 80, 64, 50, 40, 32, 16, 8):
        if n % k == 0 and k % 8 == 0:
            return k
    return n


def _accum_body(g0_ref, g1_ref, h_ref, b_ref, mc_ref, w1_ref, b1_ref,
                w2_ref, b2_ref, acc_ref, *, G, K, L, W):
    i = pl.program_id(0)
    hb = h_ref[...]
    hb16 = hb.astype(jnp.bfloat16)
    hidden = jnp.tanh(
        jnp.dot(hb16, w1_ref[...],
                preferred_element_type=jnp.float32) + b1_ref[...])
    sc = (jnp.dot(hidden.astype(jnp.bfloat16), w2_ref[...],
                  preferred_element_type=jnp.float32)
          + b2_ref[...])
    mcol = mc_ref[...]
    brow = b_ref[0]

    e_cols = [jnp.exp(sc[:, j:j + 1]).astype(jnp.bfloat16) for j in range(4)]
    m16 = mcol.astype(jnp.bfloat16)
    pieces = [ec * hb16 for ec in e_cols] + [m16 * hb16] + e_cols + [m16]
    data = jnp.concatenate(pieces, axis=1)

    bmin = g0_ref[i]
    bmax = g1_ref[i]
    g0a = jnp.minimum((bmin // 8) * 8, G - L)
    ok = bmax < g0a + L

    @pl.when(i == 0)
    def _init():
        acc_ref[...] = jnp.zeros((G, W), jnp.float32)

    @pl.when(ok)
    def _local():
        lcol = jax.lax.broadcasted_iota(jnp.int32, (L, 1), 0) + g0a
        oh = (lcol == brow)
        contrib = jnp.dot(oh.astype(jnp.bfloat16), data,
                          preferred_element_type=jnp.float32)
        acc_ref[pl.ds(g0a, L), :] += contrib

    @pl.when(jnp.logical_not(ok))
    def _full():
        gcol = jax.lax.broadcasted_iota(jnp.int32, (G, 1), 0)
        oh = (gcol == brow)
        contrib = jnp.dot(oh.astype(jnp.bfloat16), data,
                          preferred_element_type=jnp.float32)
        acc_ref[...] += contrib


def _finish_body(acc_ref, dp_ref, pW_ref, pb_ref, de_ref, fW1_ref, fb1_ref,
                 fW2_ref, fb2_ref, g_ref, be_ref, out_ref, *, G, D):
    acc = acc_ref[...]
    heads = []
    for j in range(4):
        num = acc[:, j * D:(j + 1) * D]
        den = acc[:, 5 * D + j:5 * D + j + 1]
        den = jnp.where(den == 0.0, 1.0, den)
        heads.append(num / den)
    hcat = jnp.concatenate(heads, axis=1)
    hg = jnp.dot(hcat, pW_ref[...],
                 preferred_element_type=jnp.float32) + pb_ref[...]
    hb = acc[:, 4 * D:5 * D] / (acc[:, 5 * D + 4:5 * D + 5] + 1e-8)

    mdrow = jnp.max(dp_ref[...], axis=0, keepdims=True)
    mdrow = jnp.clip(jnp.maximum(mdrow, 0), 0, MAX_DEPTH - 1)
    kcol = jax.lax.broadcasted_iota(jnp.int32, (MAX_DEPTH, 1), 0)
    ohdT = (kcol == mdrow).astype(jnp.float32)
    denc = jax.lax.dot_general(
        ohdT, de_ref[...], (((0,), (0,)), ((), ())),
        preferred_element_type=jnp.float32)

    fused = jnp.concatenate([hg, hb, denc], axis=1)
    x = jnp.dot(fused, fW1_ref[...],
                preferred_element_type=jnp.float32) + fb1_ref[...]
    x = 0.5 * x * (1.0 + jax.lax.erf(x * 0.7071067811865476))
    x = jnp.dot(x, fW2_ref[...],
                preferred_element_type=jnp.float32) + fb2_ref[...]
    mu = jnp.mean(x, axis=-1, keepdims=True)
    var = jnp.mean((x - mu) ** 2, axis=-1, keepdims=True)
    out_ref[...] = (x - mu) / jnp.sqrt(var + 1e-5) * g_ref[...] + be_ref[...]


def kernel(h, batch, is_branch, depth, aW1, ab1, aW2, ab2, pW, pb, depth_embed,
           fW1, fb1, fW2, fb2, gamma, beta):
    N, D = h.shape
    H, _, dh = aW1.shape
    G = NUM_GRAPHS
    K = _pick_block(N)
    NB = N // K
    W = 5 * D + 5

    L = 72

    w1cat = jnp.transpose(aW1, (1, 0, 2)).reshape(D, H * dh).astype(jnp.bfloat16)
    b1row = ab1.reshape(1, H * dh)
    rows = jnp.arange(H * dh)
    w2p = jnp.zeros((H * dh, 128), jnp.float32).at[rows, rows // dh].set(
        aW2.reshape(H * dh)).astype(jnp.bfloat16)
    b2row = jnp.zeros((1, 128), jnp.float32).at[0, :H].set(ab2[:, 0])

    batch_i = batch.astype(jnp.int32)
    batch_r = batch_i.reshape(NB, 1, K)
    maskc = is_branch.astype(jnp.float32).reshape(N, 1)
    g0s = batch_i[0::K]
    g1s = batch_i[K - 1::K]

    dpart = _depthmax_sc(batch_i, depth.astype(jnp.int32), G)

    acc = pl.pallas_call(
        functools.partial(_accum_body, G=G, K=K, L=L, W=W),
        grid_spec=pltpu.PrefetchScalarGridSpec(
            num_scalar_prefetch=2,
            grid=(NB,),
            in_specs=[
                pl.BlockSpec((K, D), lambda i, *_: (i, 0)),
                pl.BlockSpec((1, 1, K), lambda i, *_: (i, 0, 0)),
                pl.BlockSpec((K, 1), lambda i, *_: (i, 0)),
                pl.BlockSpec((D, H * dh), lambda i, *_: (0, 0)),
                pl.BlockSpec((1, H * dh), lambda i, *_: (0, 0)),
                pl.BlockSpec((H * dh, 128), lambda i, *_: (0, 0)),
                pl.BlockSpec((1, 128), lambda i, *_: (0, 0)),
            ],
            out_specs=[
                pl.BlockSpec((G, W), lambda i, *_: (0, 0)),
            ],
        ),
        out_shape=[
            jax.ShapeDtypeStruct((G, W), jnp.float32),
        ],
    )(g0s, g1s, h, batch_r, maskc, w1cat, b1row, w2p, b2row)[0]

    dep_p = jnp.zeros((MAX_DEPTH, 128), jnp.float32).at[:, :depth_embed.shape[1]].set(depth_embed)
    fin = 2 * D + 128
    fW1p = jnp.zeros((fin, fW1.shape[1]), jnp.float32)
    fW1p = fW1p.at[:2 * D].set(fW1[:2 * D])
    fW1p = fW1p.at[2 * D:2 * D + depth_embed.shape[1]].set(fW1[2 * D:])

    out = pl.pallas_call(
        functools.partial(_finish_body, G=G, D=D),
        in_specs=[pl.BlockSpec(x.shape, lambda: tuple(0 for _ in x.shape))
                  for x in (acc, dpart, pW, pb.reshape(1, -1), dep_p, fW1p,
                            fb1.reshape(1, -1), fW2, fb2.reshape(1, -1),
                            gamma.reshape(1, -1), beta.reshape(1, -1))],
        out_specs=pl.BlockSpec((G, fW2.shape[1]), lambda: (0, 0)),
        out_shape=jax.ShapeDtypeStruct((G, fW2.shape[1]), jnp.float32),
    )(acc, dpart, pW, pb.reshape(1, -1), dep_p, fW1p, fb1.reshape(1, -1),
      fW2, fb2.reshape(1, -1), gamma.reshape(1, -1), beta.reshape(1, -1))
    return out
